# baseline (speedup 1.0000x reference)
import functools

import jax
import jax.numpy as jnp
from jax import lax
from jax.experimental import pallas as pl
from jax.experimental.pallas import tpu as pltpu
from jax.experimental.pallas import tpu_sc as plsc

_LO = 0.3
_HI = 0.7
_NCHUNK = 8


@functools.cache
def _build(batch: int):
    info = plsc.get_sparse_core_info()
    nc, ns, lanes = info.num_cores, info.num_subcores, info.num_lanes
    nw = nc * ns
    assert batch % (8 * nw) == 0
    b_per_w = batch // nw
    nch = _NCHUNK
    assert b_per_w % (nch * lanes) == 0
    c = b_per_w // nch
    mesh = plsc.VectorSubcoreMesh(core_axis_name="c", subcore_axis_name="s")

    @functools.partial(
        pl.kernel,
        mesh=mesh,
        out_type=jax.ShapeDtypeStruct((batch,), jnp.float32),
        scratch_types=[
            pltpu.VMEM((b_per_w,), jnp.int32),
            pltpu.VMEM((b_per_w,), jnp.float32),
            pltpu.SemaphoreType.DMA((nch,)),
            pltpu.SemaphoreType.DMA((nch,)),
            pltpu.SemaphoreType.DMA((nch,)),
        ],
    )
    def gather_clip(table_hbm, idx_hbm, out_hbm, idx_v, vals_v, sem_i, sem_g, sem_s):
        wid = lax.axis_index("s") * nc + lax.axis_index("c")
        base = wid * b_per_w
        for k in range(nch):
            pltpu.async_copy(
                idx_hbm.at[pl.ds(base + k * c, c)], idx_v.at[pl.ds(k * c, c)],
                sem_i.at[k])
        for k in range(nch):
            pltpu.make_async_copy(
                idx_hbm.at[pl.ds(base + k * c, c)], idx_v.at[pl.ds(k * c, c)],
                sem_i.at[k]).wait()
            pltpu.async_copy(
                table_hbm.at[idx_v.at[pl.ds(k * c, c)]],
                vals_v.at[pl.ds(k * c, c)], sem_g.at[k])
        for k in range(nch):
            pltpu.make_async_copy(
                table_hbm.at[idx_v.at[pl.ds(k * c, c)]],
                vals_v.at[pl.ds(k * c, c)], sem_g.at[k]).wait()
            for i in range(c // lanes):
                sl = pl.ds(k * c + i * lanes, lanes)
                vals_v[sl] = jnp.minimum(jnp.maximum(vals_v[sl], _LO), _HI)
            pltpu.async_copy(
                vals_v.at[pl.ds(k * c, c)], out_hbm.at[pl.ds(base + k * c, c)],
                sem_s.at[k])
        for k in range(nch):
            pltpu.make_async_copy(
                vals_v.at[pl.ds(k * c, c)], out_hbm.at[pl.ds(base + k * c, c)],
                sem_s.at[k]).wait()

    return gather_clip


def kernel(class_sigmas, labels):
    return _build(labels.shape[0])(class_sigmas, labels)


# speedup vs baseline: 1.0241x; 1.0241x over previous
import functools

import jax
import jax.numpy as jnp
from jax import lax
from jax.experimental import pallas as pl
from jax.experimental.pallas import tpu as pltpu
from jax.experimental.pallas import tpu_sc as plsc

_LO = 0.3
_HI = 0.7
_NCHUNK = 2


@functools.cache
def _build(batch: int):
    info = plsc.get_sparse_core_info()
    nc, ns, lanes = info.num_cores, info.num_subcores, info.num_lanes
    nw = nc * ns
    assert batch % (8 * nw) == 0
    b_per_w = batch // nw
    nch = _NCHUNK
    assert b_per_w % (nch * lanes) == 0
    c = b_per_w // nch
    mesh = plsc.VectorSubcoreMesh(core_axis_name="c", subcore_axis_name="s")

    @functools.partial(
        pl.kernel,
        mesh=mesh,
        out_type=jax.ShapeDtypeStruct((batch,), jnp.float32),
        scratch_types=[
            pltpu.VMEM((b_per_w,), jnp.int32),
            pltpu.VMEM((b_per_w,), jnp.float32),
            pltpu.SemaphoreType.DMA((nch,)),
            pltpu.SemaphoreType.DMA((nch,)),
            pltpu.SemaphoreType.DMA((nch,)),
        ],
    )
    def gather_clip(table_hbm, idx_hbm, out_hbm, idx_v, vals_v, sem_i, sem_g, sem_s):
        wid = lax.axis_index("s") * nc + lax.axis_index("c")
        base = wid * b_per_w
        for k in range(nch):
            pltpu.async_copy(
                idx_hbm.at[pl.ds(base + k * c, c)], idx_v.at[pl.ds(k * c, c)],
                sem_i.at[k])
        for k in range(nch):
            pltpu.make_async_copy(
                idx_hbm.at[pl.ds(base + k * c, c)], idx_v.at[pl.ds(k * c, c)],
                sem_i.at[k]).wait()
            pltpu.async_copy(
                table_hbm.at[idx_v.at[pl.ds(k * c, c)]],
                vals_v.at[pl.ds(k * c, c)], sem_g.at[k])
        for k in range(nch):
            pltpu.make_async_copy(
                table_hbm.at[idx_v.at[pl.ds(k * c, c)]],
                vals_v.at[pl.ds(k * c, c)], sem_g.at[k]).wait()
            for i in range(c // lanes):
                sl = pl.ds(k * c + i * lanes, lanes)
                vals_v[sl] = jnp.minimum(jnp.maximum(vals_v[sl], _LO), _HI)
            pltpu.async_copy(
                vals_v.at[pl.ds(k * c, c)], out_hbm.at[pl.ds(base + k * c, c)],
                sem_s.at[k])
        for k in range(nch):
            pltpu.make_async_copy(
                vals_v.at[pl.ds(k * c, c)], out_hbm.at[pl.ds(base + k * c, c)],
                sem_s.at[k]).wait()

    return gather_clip


def kernel(class_sigmas, labels):
    return _build(labels.shape[0])(class_sigmas, labels)


# speedup vs baseline: 1.0804x; 1.0549x over previous
import functools

import jax
import jax.numpy as jnp
from jax import lax
from jax.experimental import pallas as pl
from jax.experimental.pallas import tpu as pltpu
from jax.experimental.pallas import tpu_sc as plsc

_LO = 0.3
_HI = 0.7
_NCHUNK = 2


@functools.cache
def _build(batch: int):
    info = plsc.get_sparse_core_info()
    nc, ns, lanes = 1, info.num_subcores, info.num_lanes
    nw = nc * ns
    assert batch % (8 * nw) == 0
    b_per_w = batch // nw
    nch = _NCHUNK
    assert b_per_w % (nch * lanes) == 0
    c = b_per_w // nch
    mesh = plsc.VectorSubcoreMesh(core_axis_name="c", subcore_axis_name="s",
                                  num_cores=1)

    @functools.partial(
        pl.kernel,
        mesh=mesh,
        out_type=jax.ShapeDtypeStruct((batch,), jnp.float32),
        scratch_types=[
            pltpu.VMEM((b_per_w,), jnp.int32),
            pltpu.VMEM((b_per_w,), jnp.float32),
            pltpu.SemaphoreType.DMA((nch,)),
            pltpu.SemaphoreType.DMA((nch,)),
            pltpu.SemaphoreType.DMA((nch,)),
        ],
    )
    def gather_clip(table_hbm, idx_hbm, out_hbm, idx_v, vals_v, sem_i, sem_g, sem_s):
        wid = lax.axis_index("s") * nc + lax.axis_index("c")
        base = wid * b_per_w
        for k in range(nch):
            pltpu.async_copy(
                idx_hbm.at[pl.ds(base + k * c, c)], idx_v.at[pl.ds(k * c, c)],
                sem_i.at[k])
        for k in range(nch):
            pltpu.make_async_copy(
                idx_hbm.at[pl.ds(base + k * c, c)], idx_v.at[pl.ds(k * c, c)],
                sem_i.at[k]).wait()
            pltpu.async_copy(
                table_hbm.at[idx_v.at[pl.ds(k * c, c)]],
                vals_v.at[pl.ds(k * c, c)], sem_g.at[k])
        for k in range(nch):
            pltpu.make_async_copy(
                table_hbm.at[idx_v.at[pl.ds(k * c, c)]],
                vals_v.at[pl.ds(k * c, c)], sem_g.at[k]).wait()
            for i in range(c // lanes):
                sl = pl.ds(k * c + i * lanes, lanes)
                vals_v[sl] = jnp.minimum(jnp.maximum(vals_v[sl], _LO), _HI)
            pltpu.async_copy(
                vals_v.at[pl.ds(k * c, c)], out_hbm.at[pl.ds(base + k * c, c)],
                sem_s.at[k])
        for k in range(nch):
            pltpu.make_async_copy(
                vals_v.at[pl.ds(k * c, c)], out_hbm.at[pl.ds(base + k * c, c)],
                sem_s.at[k]).wait()

    return gather_clip


def kernel(class_sigmas, labels):
    return _build(labels.shape[0])(class_sigmas, labels)
